# Initial kernel scaffold; baseline (speedup 1.0000x reference)
import jax
import jax.numpy as jnp
from jax.experimental import pallas as pl


def kernel(x, edge_index, edge_attr, W, We, be, bias):
    raise NotImplementedError("write your pallas kernel here")


# speedup vs baseline: 2.3667x; 2.3667x over previous
import jax
import jax.numpy as jnp
from jax import lax
from jax.experimental import pallas as pl
from jax.experimental.pallas import tpu as pltpu
from jax.experimental.pallas import tpu_sc as plsc

N, E, F = 10000, 320000, 128
NC, NS = 2, 16
NW = NC * NS
EPW = E // NW
CH = 80
NCH = EPW // CH
RPS = 624
TAIL = N - RPS * NS
ZB = 104


def _xt_body(x_ref, w_ref, o_ref):
    o_ref[...] = jnp.dot(x_ref[...], w_ref[...],
                         preferred_element_type=jnp.float32)


def _ew_body(ea_ref, we_ref, be_ref, o_ref):
    z = lax.dot_general(ea_ref[...], we_ref[...], (((1,), (0,)), ((), ())),
                        preferred_element_type=jnp.float32)
    o_ref[...] = jax.nn.sigmoid(z + be_ref[...])


def _combine_body(p_ref, b_ref, o_ref):
    o_ref[...] = p_ref[0] + p_ref[1] + b_ref[...]


def _sc_body(xt, ew, src, dst, outp, idx_s, idx_d, rows, ewb, zbuf, acc, sem):
    c = lax.axis_index("c")
    s = lax.axis_index("s")
    base = (c * NS + s) * EPW

    def zb_body(i, _):
        for j in range(F // 16):
            zbuf[i, pl.ds(j * 16, 16)] = jnp.zeros((16,), jnp.float32)
        return 0
    lax.fori_loop(0, ZB, zb_body, 0)
    for k in range(RPS // ZB):
        pltpu.sync_copy(zbuf, acc.at[pl.ds(s * RPS + k * ZB, ZB)])

    @pl.when(s == NS - 1)
    def _zero_tail():
        pltpu.sync_copy(zbuf.at[pl.ds(0, TAIL)], acc.at[pl.ds(RPS * NS, TAIL)])
    plsc.subcore_barrier()

    def chunk(i, _):
        off = base + i * CH
        pltpu.sync_copy(src.at[pl.ds(off, CH)], idx_s)
        pltpu.sync_copy(dst.at[pl.ds(off, CH)], idx_d)
        pltpu.sync_copy(ew.at[pl.ds(off, CH)], ewb)
        pltpu.async_copy(xt.at[idx_s], rows, sem).wait()

        def mul(e, _):
            for j in range(F // 16):
                sl = pl.ds(j * 16, 16)
                rows[e, sl] = rows[e, sl] * ewb[e, sl]
            return 0
        lax.fori_loop(0, CH, mul, 0)
        pltpu.sync_copy(rows, acc.at[idx_d], add=True)
        return 0
    lax.fori_loop(0, NCH, chunk, 0)

    plsc.subcore_barrier()
    pltpu.sync_copy(acc.at[pl.ds(s * RPS, RPS)],
                    outp.at[c, pl.ds(s * RPS, RPS)])

    @pl.when(s == NS - 1)
    def _copy_tail():
        pltpu.sync_copy(acc.at[pl.ds(RPS * NS, TAIL)],
                        outp.at[c, pl.ds(RPS * NS, TAIL)])


def kernel(x, edge_index, edge_attr, W, We, be, bias):
    xt = pl.pallas_call(
        _xt_body,
        grid=(10,),
        in_specs=[pl.BlockSpec((N // 10, F), lambda i: (i, 0)),
                  pl.BlockSpec((F, F), lambda i: (0, 0))],
        out_specs=pl.BlockSpec((N // 10, F), lambda i: (i, 0)),
        out_shape=jax.ShapeDtypeStruct((N, F), jnp.float32),
    )(x, W)

    EB = 3200
    ew = pl.pallas_call(
        _ew_body,
        grid=(E // EB,),
        in_specs=[pl.BlockSpec((EB, 4), lambda i: (i, 0)),
                  pl.BlockSpec((4, F), lambda i: (0, 0)),
                  pl.BlockSpec((1, F), lambda i: (0, 0))],
        out_specs=pl.BlockSpec((EB, F), lambda i: (i, 0)),
        out_shape=jax.ShapeDtypeStruct((E, F), jnp.float32),
    )(edge_attr, We, be.reshape(1, F))

    mesh = plsc.VectorSubcoreMesh(core_axis_name="c", subcore_axis_name="s",
                                  num_cores=NC, num_subcores=NS)
    partial = pl.kernel(
        _sc_body,
        out_type=jax.ShapeDtypeStruct((NC, N, F), jnp.float32),
        mesh=mesh,
        scratch_types=[
            pltpu.VMEM((CH,), jnp.int32),
            pltpu.VMEM((CH,), jnp.int32),
            pltpu.VMEM((CH, F), jnp.float32),
            pltpu.VMEM((CH, F), jnp.float32),
            pltpu.VMEM((ZB, F), jnp.float32),
            pltpu.VMEM_SHARED((N, F), jnp.float32),
            pltpu.SemaphoreType.DMA,
        ],
    )(xt, ew, edge_index[0], edge_index[1])

    NB = 1000
    out = pl.pallas_call(
        _combine_body,
        grid=(N // NB,),
        in_specs=[pl.BlockSpec((NC, NB, F), lambda i: (0, i, 0)),
                  pl.BlockSpec((1, F), lambda i: (0, 0))],
        out_specs=pl.BlockSpec((NB, F), lambda i: (i, 0)),
        out_shape=jax.ShapeDtypeStruct((N, F), jnp.float32),
    )(partial, bias.reshape(1, F))
    return out


# speedup vs baseline: 3.8428x; 1.6237x over previous
import jax
import jax.numpy as jnp
from jax import lax
from jax.experimental import pallas as pl
from jax.experimental.pallas import tpu as pltpu
from jax.experimental.pallas import tpu_sc as plsc

N, E, F = 10000, 320000, 128
NC, NS = 2, 16
NW = NC * NS
EPW = E // NW
CH = 40
NCH = EPW // CH
D = 2
LEAD = 1
SLACK = D - LEAD
RPS = 624
TAIL = N - RPS * NS


def _xt_body(x_ref, w_ref, o_ref):
    o_ref[...] = jnp.dot(x_ref[...], w_ref[...],
                         preferred_element_type=jnp.float32)


def _ew_body(ea_ref, we_ref, be_ref, o_ref):
    z = lax.dot_general(ea_ref[...], we_ref[...], (((1,), (0,)), ((), ())),
                        preferred_element_type=jnp.float32)
    o_ref[...] = jax.nn.sigmoid(z + be_ref[...])


def _combine_body(p_ref, b_ref, o_ref):
    o_ref[...] = p_ref[0] + p_ref[1] + b_ref[...]


def _sc_body(xt, ew, src3, dst3, outp,
             idxs, idxd, rows, ewb, acc, gsem, esem, ssem):
    c = lax.axis_index("c")
    s = lax.axis_index("s")
    w = c * NS + s

    def zb_body(i, _):
        for j in range(F // 16):
            rows[0, i, pl.ds(j * 16, 16)] = jnp.zeros((16,), jnp.float32)
        return 0
    lax.fori_loop(0, CH, zb_body, 0)
    for k in range(RPS // CH):
        pltpu.sync_copy(rows.at[0], acc.at[pl.ds(s * RPS + k * CH, CH)])
    rem = RPS - (RPS // CH) * CH
    pltpu.sync_copy(rows.at[0, pl.ds(0, rem)],
                    acc.at[pl.ds(s * RPS + (RPS // CH) * CH, rem)])

    @pl.when(s == NS - 1)
    def _zero_tail():
        pltpu.sync_copy(rows.at[0, pl.ds(0, TAIL)],
                        acc.at[pl.ds(RPS * NS, TAIL)])

    pltpu.sync_copy(src3.at[w], idxs)
    pltpu.sync_copy(dst3.at[w], idxd)
    plsc.subcore_barrier()

    def fire_fetch(i, b):
        pltpu.async_copy(xt.at[idxs.at[pl.ds(i * CH, CH)]],
                         rows.at[b], gsem.at[b])
        pltpu.async_copy(ew.at[pl.ds((w * NCH + i) * CH, CH)],
                         ewb.at[b], esem.at[b])

    for i in range(LEAD):
        fire_fetch(i, i)

    def outer(i0, _):
        for k in range(D):
            i = i0 * D + k
            bq = (k + LEAD) % D

            def drain():
                pltpu.make_async_copy(rows.at[bq],
                                      acc.at[idxd.at[pl.ds((i - SLACK) * CH,
                                                           CH)]],
                                      ssem.at[bq]).wait()
            if k >= SLACK:
                drain()
            else:
                pl.when(i0 >= 1)(drain)

            if k < D - LEAD:
                fire_fetch(i + LEAD, bq)
            else:
                @pl.when(i0 < NCH // D - 1)
                def _pf():
                    fire_fetch(i + LEAD, bq)

            pltpu.make_async_copy(xt.at[idxs.at[pl.ds(i * CH, CH)]],
                                  rows.at[k], gsem.at[k]).wait()
            pltpu.make_async_copy(ew.at[pl.ds((w * NCH + i) * CH, CH)],
                                  ewb.at[k], esem.at[k]).wait()

            def mul(e, _):
                for j in range(F // 16):
                    sl = pl.ds(j * 16, 16)
                    rows[k, e, sl] = rows[k, e, sl] * ewb[k, e, sl]
                return 0
            lax.fori_loop(0, CH, mul, 0)
            pltpu.async_copy(rows.at[k],
                             acc.at[idxd.at[pl.ds(i * CH, CH)]], ssem.at[k],
                             add=True)
        return 0
    lax.fori_loop(0, NCH // D, outer, 0)

    for i in range(NCH - SLACK, NCH):
        b = i % D
        pltpu.make_async_copy(rows.at[b],
                              acc.at[idxd.at[pl.ds(i * CH, CH)]],
                              ssem.at[b]).wait()
    plsc.subcore_barrier()

    pltpu.sync_copy(acc.at[pl.ds(s * RPS, RPS)],
                    outp.at[c, pl.ds(s * RPS, RPS)])

    @pl.when(s == NS - 1)
    def _copy_tail():
        pltpu.sync_copy(acc.at[pl.ds(RPS * NS, TAIL)],
                        outp.at[c, pl.ds(RPS * NS, TAIL)])


def kernel(x, edge_index, edge_attr, W, We, be, bias):
    xt = pl.pallas_call(
        _xt_body,
        grid=(10,),
        in_specs=[pl.BlockSpec((N // 10, F), lambda i: (i, 0)),
                  pl.BlockSpec((F, F), lambda i: (0, 0))],
        out_specs=pl.BlockSpec((N // 10, F), lambda i: (i, 0)),
        out_shape=jax.ShapeDtypeStruct((N, F), jnp.float32),
    )(x, W)

    EB = 3200
    ew = pl.pallas_call(
        _ew_body,
        grid=(E // EB,),
        in_specs=[pl.BlockSpec((EB, 4), lambda i: (i, 0)),
                  pl.BlockSpec((4, F), lambda i: (0, 0)),
                  pl.BlockSpec((1, F), lambda i: (0, 0))],
        out_specs=pl.BlockSpec((EB, F), lambda i: (i, 0)),
        out_shape=jax.ShapeDtypeStruct((E, F), jnp.float32),
    )(edge_attr, We, be.reshape(1, F))

    mesh = plsc.VectorSubcoreMesh(core_axis_name="c", subcore_axis_name="s",
                                  num_cores=NC, num_subcores=NS)
    partial = pl.kernel(
        _sc_body,
        out_type=jax.ShapeDtypeStruct((NC, N, F), jnp.float32),
        mesh=mesh,
        scratch_types=[
            pltpu.VMEM((EPW,), jnp.int32),
            pltpu.VMEM((EPW,), jnp.int32),
            pltpu.VMEM((D, CH, F), jnp.float32),
            pltpu.VMEM((D, CH, F), jnp.float32),
            pltpu.VMEM_SHARED((N, F), jnp.float32),
            pltpu.SemaphoreType.DMA((D,)),
            pltpu.SemaphoreType.DMA((D,)),
            pltpu.SemaphoreType.DMA((D,)),
        ],
    )(xt, ew, edge_index[0].reshape(NW, EPW),
      edge_index[1].reshape(NW, EPW))

    NB = 1000
    out = pl.pallas_call(
        _combine_body,
        grid=(N // NB,),
        in_specs=[pl.BlockSpec((NC, NB, F), lambda i: (0, i, 0)),
                  pl.BlockSpec((1, F), lambda i: (0, 0))],
        out_specs=pl.BlockSpec((NB, F), lambda i: (i, 0)),
        out_shape=jax.ShapeDtypeStruct((N, F), jnp.float32),
    )(partial, bias.reshape(1, F))
    return out


# speedup vs baseline: 5.0127x; 1.3044x over previous
import jax
import jax.numpy as jnp
from jax import lax
from jax.experimental import pallas as pl
from jax.experimental.pallas import tpu as pltpu
from jax.experimental.pallas import tpu_sc as plsc

N, E, F = 10000, 320000, 128
NC, NS = 2, 16
NW = NC * NS
EPW = E // NW
CH = 40
NCH = EPW // CH
D = 2
LEAD = 1
SLACK = D - LEAD
RPS = 624
TAIL = N - RPS * NS


def _xt_body(x_ref, w_ref, o_ref):
    o_ref[...] = jnp.dot(x_ref[...], w_ref[...],
                         preferred_element_type=jnp.float32)


def _ew_body(ea_ref, we_ref, be_ref, o_ref):
    z = lax.dot_general(ea_ref[...], we_ref[...], (((0,), (0,)), ((), ())),
                        preferred_element_type=jnp.float32)
    o_ref[...] = jax.nn.sigmoid(z + be_ref[...])


def _combine_body(p_ref, b_ref, o_ref):
    o_ref[...] = p_ref[0] + p_ref[1] + b_ref[...]


def _sc_body(xt, ew, src3, dst3, outp,
             idxs, idxd, rows, ewb, acc, gsem, esem, ssem):
    c = lax.axis_index("c")
    s = lax.axis_index("s")
    w = c * NS + s

    def zb_body(i, _):
        for j in range(F // 16):
            rows[0, i, pl.ds(j * 16, 16)] = jnp.zeros((16,), jnp.float32)
        return 0
    lax.fori_loop(0, CH, zb_body, 0)
    for k in range(RPS // CH):
        pltpu.sync_copy(rows.at[0], acc.at[pl.ds(s * RPS + k * CH, CH)])
    rem = RPS - (RPS // CH) * CH
    pltpu.sync_copy(rows.at[0, pl.ds(0, rem)],
                    acc.at[pl.ds(s * RPS + (RPS // CH) * CH, rem)])

    @pl.when(s == NS - 1)
    def _zero_tail():
        pltpu.sync_copy(rows.at[0, pl.ds(0, TAIL)],
                        acc.at[pl.ds(RPS * NS, TAIL)])

    pltpu.sync_copy(src3.at[w], idxs)
    pltpu.sync_copy(dst3.at[w], idxd)
    plsc.subcore_barrier()

    def fire_fetch(i, b):
        pltpu.async_copy(xt.at[idxs.at[pl.ds(i * CH, CH)]],
                         rows.at[b], gsem.at[b])
        pltpu.async_copy(ew.at[pl.ds((w * NCH + i) * CH, CH)],
                         ewb.at[b], esem.at[b])

    for i in range(LEAD):
        fire_fetch(i, i)

    def outer(i0, _):
        for k in range(D):
            i = i0 * D + k
            bq = (k + LEAD) % D

            def drain():
                pltpu.make_async_copy(rows.at[bq],
                                      acc.at[idxd.at[pl.ds((i - SLACK) * CH,
                                                           CH)]],
                                      ssem.at[bq]).wait()
            if k >= SLACK:
                drain()
            else:
                pl.when(i0 >= 1)(drain)

            if k < D - LEAD:
                fire_fetch(i + LEAD, bq)
            else:
                @pl.when(i0 < NCH // D - 1)
                def _pf():
                    fire_fetch(i + LEAD, bq)

            pltpu.make_async_copy(xt.at[idxs.at[pl.ds(i * CH, CH)]],
                                  rows.at[k], gsem.at[k]).wait()
            pltpu.make_async_copy(ew.at[pl.ds((w * NCH + i) * CH, CH)],
                                  ewb.at[k], esem.at[k]).wait()

            def mul(e, _):
                for j in range(F // 16):
                    sl = pl.ds(j * 16, 16)
                    rows[k, e, sl] = rows[k, e, sl] * ewb[k, e, sl]
                return 0
            lax.fori_loop(0, CH, mul, 0)
            pltpu.async_copy(rows.at[k],
                             acc.at[idxd.at[pl.ds(i * CH, CH)]], ssem.at[k],
                             add=True)
        return 0
    lax.fori_loop(0, NCH // D, outer, 0)

    for i in range(NCH - SLACK, NCH):
        b = i % D
        pltpu.make_async_copy(rows.at[b],
                              acc.at[idxd.at[pl.ds(i * CH, CH)]],
                              ssem.at[b]).wait()
    plsc.subcore_barrier()

    pltpu.sync_copy(acc.at[pl.ds(s * RPS, RPS)],
                    outp.at[c, pl.ds(s * RPS, RPS)])

    @pl.when(s == NS - 1)
    def _copy_tail():
        pltpu.sync_copy(acc.at[pl.ds(RPS * NS, TAIL)],
                        outp.at[c, pl.ds(RPS * NS, TAIL)])


def kernel(x, edge_index, edge_attr, W, We, be, bias):
    xt = pl.pallas_call(
        _xt_body,
        grid=(10,),
        in_specs=[pl.BlockSpec((N // 10, F), lambda i: (i, 0)),
                  pl.BlockSpec((F, F), lambda i: (0, 0))],
        out_specs=pl.BlockSpec((N // 10, F), lambda i: (i, 0)),
        out_shape=jax.ShapeDtypeStruct((N, F), jnp.float32),
    )(x, W)

    EB = 3200
    ew = pl.pallas_call(
        _ew_body,
        grid=(E // EB,),
        in_specs=[pl.BlockSpec((4, EB), lambda i: (0, i)),
                  pl.BlockSpec((4, F), lambda i: (0, 0)),
                  pl.BlockSpec((1, F), lambda i: (0, 0))],
        out_specs=pl.BlockSpec((EB, F), lambda i: (i, 0)),
        out_shape=jax.ShapeDtypeStruct((E, F), jnp.float32),
    )(edge_attr.T, We, be.reshape(1, F))

    mesh = plsc.VectorSubcoreMesh(core_axis_name="c", subcore_axis_name="s",
                                  num_cores=NC, num_subcores=NS)
    partial = pl.kernel(
        _sc_body,
        out_type=jax.ShapeDtypeStruct((NC, N, F), jnp.float32),
        mesh=mesh,
        scratch_types=[
            pltpu.VMEM((EPW,), jnp.int32),
            pltpu.VMEM((EPW,), jnp.int32),
            pltpu.VMEM((D, CH, F), jnp.float32),
            pltpu.VMEM((D, CH, F), jnp.float32),
            pltpu.VMEM_SHARED((N, F), jnp.float32),
            pltpu.SemaphoreType.DMA((D,)),
            pltpu.SemaphoreType.DMA((D,)),
            pltpu.SemaphoreType.DMA((D,)),
        ],
    )(xt, ew, edge_index[0].reshape(NW, EPW),
      edge_index[1].reshape(NW, EPW))

    NB = 1000
    out = pl.pallas_call(
        _combine_body,
        grid=(N // NB,),
        in_specs=[pl.BlockSpec((NC, NB, F), lambda i: (0, i, 0)),
                  pl.BlockSpec((1, F), lambda i: (0, 0))],
        out_specs=pl.BlockSpec((NB, F), lambda i: (i, 0)),
        out_shape=jax.ShapeDtypeStruct((N, F), jnp.float32),
    )(partial, bias.reshape(1, F))
    return out


# speedup vs baseline: 5.0149x; 1.0004x over previous
import jax
import jax.numpy as jnp
from jax import lax
from jax.experimental import pallas as pl
from jax.experimental.pallas import tpu as pltpu
from jax.experimental.pallas import tpu_sc as plsc

N, E, F = 10000, 320000, 128

NC, NS = 2, 16
NW = NC * NS
EPW = E // NW
CH = 40
NCH = EPW // CH
D = 2
LEAD = 1
SLACK = D - LEAD
RPS = 624
TAIL = N - RPS * NS


def _xt_body(x_ref, w_ref, o_ref):
    o_ref[...] = jnp.dot(x_ref[...], w_ref[...],
                         preferred_element_type=jnp.float32)


def _ew_body(ea_ref, we_ref, be_ref, o_ref):
    z = lax.dot_general(ea_ref[...], we_ref[...], (((0,), (0,)), ((), ())),
                        preferred_element_type=jnp.float32)
    o_ref[...] = jax.nn.sigmoid(z + be_ref[...])


def _combine_body(p_ref, b_ref, o_ref):
    o_ref[...] = p_ref[0] + p_ref[1] + b_ref[...]


def _sc_body(xt, ew, src3, dst3, outp,
             idxs, idxd, rows, ewb, acc, gsem, esem, ssem):
    c = lax.axis_index("c")
    s = lax.axis_index("s")
    w = c * NS + s

    def zb_body(i, _):
        for j in range(F // 16):
            rows[0, i, pl.ds(j * 16, 16)] = jnp.zeros((16,), jnp.float32)
        return 0
    lax.fori_loop(0, CH, zb_body, 0)
    for k in range(RPS // CH):
        pltpu.sync_copy(rows.at[0], acc.at[pl.ds(s * RPS + k * CH, CH)])
    rem = RPS - (RPS // CH) * CH
    pltpu.sync_copy(rows.at[0, pl.ds(0, rem)],
                    acc.at[pl.ds(s * RPS + (RPS // CH) * CH, rem)])

    @pl.when(s == NS - 1)
    def _zero_tail():
        pltpu.sync_copy(rows.at[0, pl.ds(0, TAIL)],
                        acc.at[pl.ds(RPS * NS, TAIL)])

    pltpu.sync_copy(src3.at[w], idxs)
    pltpu.sync_copy(dst3.at[w], idxd)
    plsc.subcore_barrier()

    def fire_fetch(i, b):
        pltpu.async_copy(xt.at[idxs.at[pl.ds(i * CH, CH)]],
                         rows.at[b], gsem.at[b])
        pltpu.async_copy(ew.at[pl.ds((w * NCH + i) * CH * F, CH * F)],
                         ewb.at[pl.ds(b * CH * F, CH * F)], esem.at[b])

    for i in range(LEAD):
        fire_fetch(i, i)

    def outer(i0, _):
        for k in range(D):
            i = i0 * D + k
            bq = (k + LEAD) % D

            def drain():
                pltpu.make_async_copy(rows.at[bq],
                                      acc.at[idxd.at[pl.ds((i - SLACK) * CH,
                                                           CH)]],
                                      ssem.at[bq]).wait()
            if k >= SLACK:
                drain()
            else:
                pl.when(i0 >= 1)(drain)

            if k < D - LEAD:
                fire_fetch(i + LEAD, bq)
            else:
                @pl.when(i0 < NCH // D - 1)
                def _pf():
                    fire_fetch(i + LEAD, bq)

            pltpu.make_async_copy(xt.at[idxs.at[pl.ds(i * CH, CH)]],
                                  rows.at[k], gsem.at[k]).wait()
            pltpu.make_async_copy(ew.at[pl.ds((w * NCH + i) * CH * F, CH * F)],
                                  ewb.at[pl.ds(k * CH * F, CH * F)],
                                  esem.at[k]).wait()

            def mul(e, _):
                base = k * CH * F + e * F
                for g in range(F // 16):
                    sl = pl.ds(g * 16, 16)
                    rows[k, e, sl] = rows[k, e, sl] * ewb[pl.ds(base + g * 16,
                                                               16)]
                return 0
            lax.fori_loop(0, CH, mul, 0)
            pltpu.async_copy(rows.at[k],
                             acc.at[idxd.at[pl.ds(i * CH, CH)]], ssem.at[k],
                             add=True)
        return 0
    lax.fori_loop(0, NCH // D, outer, 0)

    for i in range(NCH - SLACK, NCH):
        b = i % D
        pltpu.make_async_copy(rows.at[b],
                              acc.at[idxd.at[pl.ds(i * CH, CH)]],
                              ssem.at[b]).wait()
    plsc.subcore_barrier()

    pltpu.sync_copy(acc.at[pl.ds(s * RPS, RPS)],
                    outp.at[c, pl.ds(s * RPS, RPS)])

    @pl.when(s == NS - 1)
    def _copy_tail():
        pltpu.sync_copy(acc.at[pl.ds(RPS * NS, TAIL)],
                        outp.at[c, pl.ds(RPS * NS, TAIL)])


def kernel(x, edge_index, edge_attr, W, We, be, bias):
    xt = pl.pallas_call(
        _xt_body,
        grid=(10,),
        in_specs=[pl.BlockSpec((N // 10, F), lambda i: (i, 0)),
                  pl.BlockSpec((F, F), lambda i: (0, 0))],
        out_specs=pl.BlockSpec((N // 10, F), lambda i: (i, 0)),
        out_shape=jax.ShapeDtypeStruct((N, F), jnp.float32),
    )(x, W)

    EB = 3200
    ew = pl.pallas_call(
        _ew_body,
        grid=(E // EB,),
        in_specs=[pl.BlockSpec((4, EB), lambda i: (0, i)),
                  pl.BlockSpec((4, F), lambda i: (0, 0)),
                  pl.BlockSpec((1, F), lambda i: (0, 0))],
        out_specs=pl.BlockSpec((EB, F), lambda i: (i, 0)),
        out_shape=jax.ShapeDtypeStruct((E, F), jnp.float32),
    )(edge_attr.T, We, be.reshape(1, F))

    mesh = plsc.VectorSubcoreMesh(core_axis_name="c", subcore_axis_name="s",
                                  num_cores=NC, num_subcores=NS)
    partial = pl.kernel(
        _sc_body,
        out_type=jax.ShapeDtypeStruct((NC, N, F), jnp.float32),
        mesh=mesh,
        scratch_types=[
            pltpu.VMEM((EPW,), jnp.int32),
            pltpu.VMEM((EPW,), jnp.int32),
            pltpu.VMEM((D, CH, F), jnp.float32),
            pltpu.VMEM((D * CH * F,), jnp.float32),
            pltpu.VMEM_SHARED((N, F), jnp.float32),
            pltpu.SemaphoreType.DMA((D,)),
            pltpu.SemaphoreType.DMA((D,)),
            pltpu.SemaphoreType.DMA((D,)),
        ],
    )(xt, ew.reshape(E * F), edge_index[0].reshape(NW, EPW),
      edge_index[1].reshape(NW, EPW))

    NB = 1000
    out = pl.pallas_call(
        _combine_body,
        grid=(N // NB,),
        in_specs=[pl.BlockSpec((NC, NB, F), lambda i: (0, i, 0)),
                  pl.BlockSpec((1, F), lambda i: (0, 0))],
        out_specs=pl.BlockSpec((NB, F), lambda i: (i, 0)),
        out_shape=jax.ShapeDtypeStruct((N, F), jnp.float32),
    )(partial, bias.reshape(1, F))
    return out
